# baseline (speedup 1.0000x reference)
import functools

import jax
import jax.numpy as jnp
from jax import lax
from jax.experimental import pallas as pl
from jax.experimental.pallas import tpu as pltpu
from jax.experimental.pallas import tpu_sc as plsc

B = 1024
S = 200
D = 128
NC = 2
NS = 16
NW = NC * NS
SEQ_PER_W = B // NW
IDX_CHUNKS = 2
IDX_CHUNK = S // IDX_CHUNKS
NBUF = 4


def _emb_body(x_hbm, word_hbm, pos_hbm, out_hbm, idx_v, rows_v, sems):
    wid = lax.axis_index("s") * NC + lax.axis_index("c")
    base = wid * SEQ_PER_W

    pltpu.sync_copy(x_hbm.at[pl.ds(base, SEQ_PER_W)], idx_v)

    def prefill(seq_off, b):
        return pltpu.async_copy(
            pos_hbm.at[pl.ds(0, S)], rows_v.at[b], sems.at[b]
        )

    def gathers(seq_off, b):
        return [
            pltpu.async_copy(
                word_hbm.at[idx_v.at[seq_off, j]],
                rows_v.at[b, pl.ds(j * IDX_CHUNK, IDX_CHUNK)],
                sems.at[b],
                add=True,
            )
            for j in range(IDX_CHUNKS)
        ]

    def writeout(seq_off, b):
        return pltpu.async_copy(
            rows_v.at[b], out_hbm.at[base + seq_off], sems.at[b]
        )

    pending = {}
    for b in range(NBUF):
        pending[("p", b)] = prefill(b, b)
    for i in range(SEQ_PER_W + 1):
        if 4 <= i + 2 < SEQ_PER_W:
            b = (i + 2) % NBUF
            pending.pop(("o", b)).wait()
            pending[("p", b)] = prefill(i + 2, b)
        if i < SEQ_PER_W:
            b = i % NBUF
            pending.pop(("p", b)).wait()
            pending[("g", b)] = gathers(i, b)
        if 1 <= i:
            b = (i - 1) % NBUF
            for g in pending.pop(("g", b)):
                g.wait()
            pending[("o", b)] = writeout(i - 1, b)
    for b in range(NBUF):
        o = pending.pop(("o", b), None)
        if o is not None:
            o.wait()


@functools.cache
def _make_emb_kernel():
    return pl.kernel(
        _emb_body,
        out_type=jax.ShapeDtypeStruct((B, S, D), jnp.float32),
        mesh=plsc.VectorSubcoreMesh(
            core_axis_name="c", subcore_axis_name="s", num_cores=NC, num_subcores=NS
        ),
        scratch_types=[
            pltpu.VMEM((SEQ_PER_W, IDX_CHUNKS, IDX_CHUNK), jnp.int32),
            pltpu.VMEM((NBUF, S, D), jnp.float32),
            pltpu.SemaphoreType.DMA((NBUF,)),
        ],
    )


@jax.jit
def kernel(x, word_emb, pos_emb):
    x = x.reshape(B, IDX_CHUNKS, IDX_CHUNK).astype(jnp.int32)
    return _make_emb_kernel()(x, word_emb, pos_emb)


# speedup vs baseline: 1.4283x; 1.4283x over previous
import functools

import jax
import jax.numpy as jnp
from jax import lax
from jax.experimental import pallas as pl
from jax.experimental.pallas import tpu as pltpu
from jax.experimental.pallas import tpu_sc as plsc

B = 1024
S = 200
D = 128
L = 16
NC = 2
NS = 16
NW = NC * NS
SEQ_PER_W = B // NW
IDX_CHUNKS = 2
IDX_CHUNK = S // IDX_CHUNKS
NBUF = 3


def _emb_body(x_hbm, word_hbm, pos_hbm, out_hbm, idx_v, rows_v, pos_v, sems):
    wid = lax.axis_index("s") * NC + lax.axis_index("c")
    base = wid * SEQ_PER_W

    pltpu.sync_copy(x_hbm.at[pl.ds(base, SEQ_PER_W)], idx_v)
    pltpu.sync_copy(pos_hbm.at[pl.ds(0, S)], pos_v)

    def gathers(seq_off, b):
        return [
            pltpu.async_copy(
                word_hbm.at[idx_v.at[seq_off, j]],
                rows_v.at[b, pl.ds(j * IDX_CHUNK, IDX_CHUNK)],
                sems.at[b],
            )
            for j in range(IDX_CHUNKS)
        ]

    def writeout(seq_off, b):
        return pltpu.async_copy(
            rows_v.at[b], out_hbm.at[base + seq_off], sems.at[b]
        )

    def add_pos(b):
        def add_row(r, carry):
            for d in range(D // L):
                sl = pl.ds(d * L, L)
                rows_v[b, r, sl] = rows_v[b, r, sl] + pos_v[r, sl]
            return carry

        lax.fori_loop(0, S, add_row, 0, unroll=4)

    pending = {}
    for i0 in range(2):
        pending[("g", i0 % NBUF)] = gathers(i0, i0 % NBUF)
    for i in range(SEQ_PER_W):
        if i + 2 < SEQ_PER_W:
            b = (i + 2) % NBUF
            if i + 2 >= NBUF:
                pending.pop(("o", b)).wait()
            pending[("g", b)] = gathers(i + 2, b)
        b = i % NBUF
        for g in pending.pop(("g", b)):
            g.wait()
        add_pos(b)
        pending[("o", b)] = writeout(i, b)
    for b in range(NBUF):
        o = pending.pop(("o", b), None)
        if o is not None:
            o.wait()


@functools.cache
def _make_emb_kernel():
    return pl.kernel(
        _emb_body,
        out_type=jax.ShapeDtypeStruct((B, S, D), jnp.float32),
        mesh=plsc.VectorSubcoreMesh(
            core_axis_name="c", subcore_axis_name="s", num_cores=NC, num_subcores=NS
        ),
        scratch_types=[
            pltpu.VMEM((SEQ_PER_W, IDX_CHUNKS, IDX_CHUNK), jnp.int32),
            pltpu.VMEM((NBUF, S, D), jnp.float32),
            pltpu.VMEM((S, D), jnp.float32),
            pltpu.SemaphoreType.DMA((NBUF,)),
        ],
    )


@jax.jit
def kernel(x, word_emb, pos_emb):
    x = x.reshape(B, IDX_CHUNKS, IDX_CHUNK).astype(jnp.int32)
    return _make_emb_kernel()(x, word_emb, pos_emb)


# speedup vs baseline: 2.6166x; 1.8320x over previous
import functools

import jax
import jax.numpy as jnp
from jax import lax
from jax.experimental import pallas as pl
from jax.experimental.pallas import tpu as pltpu
from jax.experimental.pallas import tpu_sc as plsc

B = 1024
S = 200
D = 128
NC = 2
NS = 16
NW = NC * NS
SEQ_PER_W = B // NW
IDX_CHUNKS = 2
IDX_CHUNK = S // IDX_CHUNKS
NBUF = 4


def _emb_body(x_hbm, word_hbm, pos_hbm, out_hbm, idx_v, rows_v, pos_sh, sems):
    sid = lax.axis_index("s")
    wid = sid * NC + lax.axis_index("c")
    base = wid * SEQ_PER_W

    pltpu.sync_copy(x_hbm.at[pl.ds(base, SEQ_PER_W)], idx_v)
    @pl.when(sid == 0)
    def _stage_pos():
        pltpu.sync_copy(pos_hbm.at[pl.ds(0, S)], rows_v.at[0])
        pltpu.sync_copy(rows_v.at[0], pos_sh)

    plsc.subcore_barrier()

    def prefill(b):
        return pltpu.async_copy(pos_sh, rows_v.at[b], sems.at[b])

    def gathers(seq_off, b):
        return [
            pltpu.async_copy(
                word_hbm.at[idx_v.at[seq_off, j]],
                rows_v.at[b, pl.ds(j * IDX_CHUNK, IDX_CHUNK)],
                sems.at[b],
                add=True,
            )
            for j in range(IDX_CHUNKS)
        ]

    def writeout(seq_off, b):
        return pltpu.async_copy(
            rows_v.at[b], out_hbm.at[base + seq_off], sems.at[b]
        )

    pending = {}
    for b in range(NBUF):
        pending[("p", b)] = prefill(b)
    for i in range(SEQ_PER_W + 1):
        if 4 <= i + 2 < SEQ_PER_W:
            b = (i + 2) % NBUF
            pending.pop(("o", b)).wait()
            pending[("p", b)] = prefill(b)
        if i < SEQ_PER_W:
            b = i % NBUF
            pending.pop(("p", b)).wait()
            pending[("g", b)] = gathers(i, b)
        if 1 <= i:
            b = (i - 1) % NBUF
            for g in pending.pop(("g", b)):
                g.wait()
            pending[("o", b)] = writeout(i - 1, b)
    for b in range(NBUF):
        o = pending.pop(("o", b), None)
        if o is not None:
            o.wait()


@functools.cache
def _make_emb_kernel():
    return pl.kernel(
        _emb_body,
        out_type=jax.ShapeDtypeStruct((B, S, D), jnp.float32),
        mesh=plsc.VectorSubcoreMesh(
            core_axis_name="c", subcore_axis_name="s", num_cores=NC, num_subcores=NS
        ),
        scratch_types=[
            pltpu.VMEM((SEQ_PER_W, IDX_CHUNKS, IDX_CHUNK), jnp.int32),
            pltpu.VMEM((NBUF, S, D), jnp.float32),
            pltpu.VMEM_SHARED((S, D), jnp.float32),
            pltpu.SemaphoreType.DMA((NBUF,)),
        ],
    )


@jax.jit
def kernel(x, word_emb, pos_emb):
    x = x.reshape(B, IDX_CHUNKS, IDX_CHUNK).astype(jnp.int32)
    return _make_emb_kernel()(x, word_emb, pos_emb)
